# initial kernel scaffold (unmeasured)
import jax
import jax.numpy as jnp
from jax import lax
from jax.experimental import pallas as pl
from jax.experimental.pallas import tpu as pltpu

N_DEV = 32
SQ = 512
D = 1024
H_LOC = 8
DH = 128
SCALE = 0.08838834764831843


def _neighbor_barrier(left, right):
    barrier = pltpu.get_barrier_semaphore()
    for nbr in (left, right):
        pl.semaphore_signal(
            barrier, inc=1, device_id=(nbr,),
            device_id_type=pl.DeviceIdType.MESH,
        )
    pl.semaphore_wait(barrier, 2)


def _ring_allgather(x_bf16):

    def body(x_ref, out_ref, send_sems, recv_sems):
        me = lax.axis_index("i")
        left = lax.rem(me + N_DEV - 1, N_DEV)
        right = lax.rem(me + 1, N_DEV)
        _neighbor_barrier(left, right)

        out_ref[pl.ds(me, 1)] = x_ref[...][None]
        for h in range(N_DEV - 1):
            src = lax.rem(me - h + 2 * N_DEV, N_DEV)
            rdma = pltpu.make_async_remote_copy(
                src_ref=out_ref.at[src],
                dst_ref=out_ref.at[src],
                send_sem=send_sems.at[h],
                recv_sem=recv_sems.at[h],
                device_id=(right,),
                device_id_type=pl.DeviceIdType.MESH,
            )
            rdma.start()
            rdma.wait()

    return pl.pallas_call(
        body,
        out_shape=jax.ShapeDtypeStruct((N_DEV, SQ, D), jnp.bfloat16),
        in_specs=[pl.BlockSpec(memory_space=pltpu.VMEM)],
        out_specs=pl.BlockSpec(memory_space=pltpu.VMEM),
        scratch_shapes=[
            pltpu.SemaphoreType.DMA((N_DEV - 1,)),
            pltpu.SemaphoreType.DMA((N_DEV - 1,)),
        ],
        compiler_params=pltpu.CompilerParams(collective_id=0),
    )(x_bf16)


def _local_attention(xg, Wq, Wk, Wv, Wo):
    wq = Wq.astype(jnp.bfloat16)
    wk = Wk.astype(jnp.bfloat16)
    wv = Wv.astype(jnp.bfloat16)
    wo = Wo.astype(jnp.bfloat16)

    def proj(w):
        p = jnp.einsum(
            "bsd,de->bse", xg, w, preferred_element_type=jnp.float32
        )
        return p.astype(jnp.bfloat16).reshape(N_DEV, SQ, H_LOC, DH)

    q, k, v = proj(wq), proj(wk), proj(wv)
    s = jnp.einsum(
        "bihd,bjhd->bhij", q, k, preferred_element_type=jnp.float32
    ) * SCALE
    m = s.max(axis=-1, keepdims=True)
    p = jnp.exp(s - m)
    l = p.sum(axis=-1, keepdims=True)
    o = jnp.einsum(
        "bhij,bjhd->bihd",
        p.astype(jnp.bfloat16),
        v,
        preferred_element_type=jnp.float32,
    )
    o = o / l.transpose(0, 2, 1, 3)
    o = o.reshape(N_DEV, SQ, H_LOC * DH).astype(jnp.bfloat16)
    return jnp.einsum(
        "bse,ed->bsd", o, wo, preferred_element_type=jnp.float32
    )


def _ring_reduce_scatter(partial):

    def body(p_ref, out_ref, vsend, vrecv, pbuf, send_sems, recv_sems,
             local_sem, credit_sem):
        me = lax.axis_index("i")
        left = lax.rem(me + N_DEV - 1, N_DEV)
        right = lax.rem(me + 1, N_DEV)
        _neighbor_barrier(left, right)

        c0 = lax.rem(me + N_DEV - 1, N_DEV)
        cp = pltpu.make_async_copy(p_ref.at[c0], vsend.at[0], local_sem)
        cp.start()
        cp.wait()

        for h in range(N_DEV - 1):
            if h >= 2:
                pl.semaphore_wait(credit_sem, 1)
            rdma = pltpu.make_async_remote_copy(
                src_ref=vsend.at[h % 2],
                dst_ref=vrecv.at[h % 2],
                send_sem=send_sems.at[h],
                recv_sem=recv_sems.at[h],
                device_id=(right,),
                device_id_type=pl.DeviceIdType.MESH,
            )
            rdma.start()
            rdma.wait()

            cin = lax.rem(me - 2 - h + 2 * N_DEV, N_DEV)
            cp = pltpu.make_async_copy(p_ref.at[cin], pbuf, local_sem)
            cp.start()
            cp.wait()
            if h < N_DEV - 2:
                vsend[(h + 1) % 2] = vrecv[h % 2] + pbuf[...]
                if h <= N_DEV - 4:
                    pl.semaphore_signal(
                        credit_sem, inc=1, device_id=(left,),
                        device_id_type=pl.DeviceIdType.MESH,
                    )
            else:
                out_ref[...] = vrecv[h % 2] + pbuf[...]

    return pl.pallas_call(
        body,
        out_shape=jax.ShapeDtypeStruct((SQ, D), jnp.float32),
        in_specs=[pl.BlockSpec(memory_space=pltpu.ANY)],
        out_specs=pl.BlockSpec(memory_space=pltpu.VMEM),
        scratch_shapes=[
            pltpu.VMEM((2, SQ, D), jnp.float32),
            pltpu.VMEM((2, SQ, D), jnp.float32),
            pltpu.VMEM((SQ, D), jnp.float32),
            pltpu.SemaphoreType.DMA((N_DEV - 1,)),
            pltpu.SemaphoreType.DMA((N_DEV - 1,)),
            pltpu.SemaphoreType.DMA,
            pltpu.SemaphoreType.REGULAR,
        ],
        compiler_params=pltpu.CompilerParams(collective_id=1),
    )(partial)


def kernel(x, Wq, Wo, Wk, Wv):
    x = x.reshape(SQ, D).astype(jnp.bfloat16)
    xg = _ring_allgather(x)
    partial = _local_attention(xg, Wq, Wk, Wv, Wo)
    out = _ring_reduce_scatter(partial)
    return out.reshape(1, SQ, D)


# baseline (device time: 1719344 ns/iter reference)
import jax
import jax.numpy as jnp
from jax import lax
from jax.experimental import pallas as pl
from jax.experimental.pallas import tpu as pltpu

N_DEV = 32
SQ = 512
D = 1024
H_LOC = 8
DH = 128
SCALE = 0.08838834764831843


def _neighbor_barrier(left, right):
    barrier = pltpu.get_barrier_semaphore()
    for nbr in (left, right):
        pl.semaphore_signal(
            barrier, inc=1, device_id=(nbr,),
            device_id_type=pl.DeviceIdType.MESH,
        )
    pl.semaphore_wait(barrier, 2)


def _ring_allgather(x_bf16):

    def body(x_ref, out_ref, send_sems, recv_sems):
        me = lax.axis_index("i")
        left = lax.rem(me + N_DEV - 1, N_DEV)
        right = lax.rem(me + 1, N_DEV)
        _neighbor_barrier(left, right)

        out_ref[pl.ds(me, 1)] = x_ref[...][None]
        for h in range(N_DEV - 1):
            src = lax.rem(me - h + 2 * N_DEV, N_DEV)
            rdma = pltpu.make_async_remote_copy(
                src_ref=out_ref.at[src],
                dst_ref=out_ref.at[src],
                send_sem=send_sems.at[h],
                recv_sem=recv_sems.at[h],
                device_id=(right,),
                device_id_type=pl.DeviceIdType.MESH,
            )
            rdma.start()
            rdma.wait()

    return pl.pallas_call(
        body,
        out_shape=jax.ShapeDtypeStruct((N_DEV, SQ, D), jnp.bfloat16),
        in_specs=[pl.BlockSpec(memory_space=pltpu.VMEM)],
        out_specs=pl.BlockSpec(memory_space=pltpu.VMEM),
        scratch_shapes=[
            pltpu.SemaphoreType.DMA((N_DEV - 1,)),
            pltpu.SemaphoreType.DMA((N_DEV - 1,)),
        ],
        compiler_params=pltpu.CompilerParams(collective_id=0),
    )(x_bf16)


def _local_attention(xg, Wq, Wk, Wv, Wo):
    wq = Wq.astype(jnp.bfloat16)
    wk = Wk.astype(jnp.bfloat16)
    wv = Wv.astype(jnp.bfloat16)
    wo = Wo.astype(jnp.bfloat16)

    def proj(w):
        p = jnp.einsum(
            "bsd,de->bse", xg, w, preferred_element_type=jnp.float32
        )
        return p.astype(jnp.bfloat16).reshape(N_DEV, SQ, H_LOC, DH)

    q, k, v = proj(wq), proj(wk), proj(wv)
    s = jnp.einsum(
        "bihd,bjhd->bhij", q, k, preferred_element_type=jnp.float32
    ) * SCALE
    m = s.max(axis=-1, keepdims=True)
    p = jnp.exp(s - m)
    l = p.sum(axis=-1, keepdims=True)
    o = jnp.einsum(
        "bhij,bjhd->bihd",
        p.astype(jnp.bfloat16),
        v,
        preferred_element_type=jnp.float32,
    )
    o = o / l.transpose(0, 2, 1, 3)
    o = o.reshape(N_DEV, SQ, H_LOC * DH).astype(jnp.bfloat16)
    return jnp.einsum(
        "bse,ed->bsd", o, wo, preferred_element_type=jnp.float32
    )


def _ring_reduce_scatter(partial):

    def body(p_ref, out_ref, vsend, vrecv, pbuf, send_sems, recv_sems,
             local_sem, credit_sem):
        me = lax.axis_index("i")
        left = lax.rem(me + N_DEV - 1, N_DEV)
        right = lax.rem(me + 1, N_DEV)
        _neighbor_barrier(left, right)

        c0 = lax.rem(me + N_DEV - 1, N_DEV)
        cp = pltpu.make_async_copy(p_ref.at[c0], vsend.at[0], local_sem)
        cp.start()
        cp.wait()

        for h in range(N_DEV - 1):
            if h >= 2:
                pl.semaphore_wait(credit_sem, 1)
            rdma = pltpu.make_async_remote_copy(
                src_ref=vsend.at[h % 2],
                dst_ref=vrecv.at[h % 2],
                send_sem=send_sems.at[h],
                recv_sem=recv_sems.at[h],
                device_id=(right,),
                device_id_type=pl.DeviceIdType.MESH,
            )
            rdma.start()
            rdma.wait()

            cin = lax.rem(me - 2 - h + 2 * N_DEV, N_DEV)
            cp = pltpu.make_async_copy(p_ref.at[cin], pbuf, local_sem)
            cp.start()
            cp.wait()
            if h < N_DEV - 2:
                vsend[(h + 1) % 2] = vrecv[h % 2] + pbuf[...]
                if h <= N_DEV - 4:
                    pl.semaphore_signal(
                        credit_sem, inc=1, device_id=(left,),
                        device_id_type=pl.DeviceIdType.MESH,
                    )
            else:
                out_ref[...] = vrecv[h % 2] + pbuf[...]

    return pl.pallas_call(
        body,
        out_shape=jax.ShapeDtypeStruct((SQ, D), jnp.float32),
        in_specs=[pl.BlockSpec(memory_space=pl.ANY)],
        out_specs=pl.BlockSpec(memory_space=pltpu.VMEM),
        scratch_shapes=[
            pltpu.VMEM((2, SQ, D), jnp.float32),
            pltpu.VMEM((2, SQ, D), jnp.float32),
            pltpu.VMEM((SQ, D), jnp.float32),
            pltpu.SemaphoreType.DMA((N_DEV - 1,)),
            pltpu.SemaphoreType.DMA((N_DEV - 1,)),
            pltpu.SemaphoreType.DMA,
            pltpu.SemaphoreType.REGULAR,
        ],
        compiler_params=pltpu.CompilerParams(collective_id=1),
    )(partial)


def kernel(x, Wq, Wo, Wk, Wv):
    x = x.reshape(SQ, D).astype(jnp.bfloat16)
    xg = _ring_allgather(x)
    partial = _local_attention(xg, Wq, Wk, Wv, Wo)
    out = _ring_reduce_scatter(partial)
    return out.reshape(1, SQ, D)
